# Initial kernel scaffold; baseline (speedup 1.0000x reference)
import jax
import jax.numpy as jnp
from jax.experimental import pallas as pl


def kernel(x, target):
    raise NotImplementedError("write your pallas kernel here")


# speedup vs baseline: 7.6152x; 7.6152x over previous
import functools
import math

import jax
import jax.numpy as jnp
from jax.experimental import pallas as pl
from jax.experimental.pallas import tpu as pltpu

_PAD = 0
_SMOOTHING = 0.1
_CONFIDENCE = 1.0 - _SMOOTHING


def _loss_body(tgt_ref, x_ref, out_ref, acc_ref, nv_ref, *, ncb, cb, smooth):
    j = pl.program_id(0)
    t = tgt_ref[...]
    valid = (t != _PAD)
    validf = valid.astype(jnp.float32)
    xb = x_ref[...]
    n, _ = xb.shape

    rs = jnp.sum(xb, axis=1, keepdims=True)
    cols = jax.lax.broadcasted_iota(jnp.int32, (n, cb), 1) + j * cb
    g = jnp.sum(jnp.where(cols == t, xb, 0.0), axis=1, keepdims=True)

    contrib = jnp.sum(validf * (-smooth * rs + (smooth - _CONFIDENCE) * g))

    @pl.when(j == 0)
    def _init():
        nv = jnp.sum(validf)
        nv_ref[0] = nv
        x0 = xb[:, 0:1]
        acc_ref[0] = contrib + smooth * jnp.sum(validf * x0)

    @pl.when(j > 0)
    def _accum():
        acc_ref[0] = acc_ref[0] + contrib

    @pl.when(j == ncb - 1)
    def _finish():
        v = x_ref.shape[1] * ncb
        k_const = ((v - 2) * smooth * math.log(smooth)
                   + _CONFIDENCE * math.log(_CONFIDENCE))
        out_ref[0, 0] = (acc_ref[0] + k_const * nv_ref[0]) / nv_ref[0]


def kernel(x, target):
    x2 = x.reshape(-1, x.shape[-1])
    n, v = x2.shape
    tgt = target.reshape(-1, 1).astype(jnp.int32)
    cb = 512
    ncb = v // cb
    smooth = _SMOOTHING / (v - 2)

    out = pl.pallas_call(
        functools.partial(_loss_body, ncb=ncb, cb=cb, smooth=smooth),
        grid=(ncb,),
        in_specs=[
            pl.BlockSpec((n, 1), lambda j: (0, 0)),
            pl.BlockSpec((n, cb), lambda j: (0, j)),
        ],
        out_specs=pl.BlockSpec(memory_space=pltpu.SMEM),
        out_shape=jax.ShapeDtypeStruct((1, 1), jnp.float32),
        scratch_shapes=[
            pltpu.SMEM((1,), jnp.float32),
            pltpu.SMEM((1,), jnp.float32),
        ],
    )(tgt, x2)
    return out[0, 0]
